# Initial kernel scaffold; baseline (speedup 1.0000x reference)
import functools
import math

import jax
import jax.numpy as jnp
from jax import lax
from jax.experimental import pallas as pl
from jax.experimental.pallas import tpu as pltpu
from jax.experimental.pallas import tpu_sc as plsc

H = 8


def _qkv_body(xb, wq, wk, wv, bq, bk, bv, qo, ko, vo):
    x = xb[...]
    dn = (((1,), (1,)), ((), ()))
    qo[...] = lax.dot_general(x, wq[...], dn, preferred_element_type=jnp.float32) + bq[...]
    ko[...] = lax.dot_general(x, wk[...], dn, preferred_element_type=jnp.float32) + bk[...]
    vo[...] = lax.dot_general(x, wv[...], dn, preferred_element_type=jnp.float32) + bv[...]


def _qkv_proj(x, Wq, bq, Wk, bk, Wv, bv):
    N, D = x.shape
    Dh = D // 2
    B = 625
    nb = N // B
    out_sd = jax.ShapeDtypeStruct((2 * N, Dh), jnp.float32)
    w_spec = pl.BlockSpec((Dh, D), lambda i, h: (h, 0))
    b_spec = pl.BlockSpec((1, Dh), lambda i, h: (h, 0))
    o_spec = pl.BlockSpec((B, Dh), lambda i, h: (h * nb + i, 0))
    return pl.pallas_call(
        _qkv_body,
        grid=(nb, 2),
        in_specs=[
            pl.BlockSpec((B, D), lambda i, h: (i, 0)),
            w_spec, w_spec, w_spec, b_spec, b_spec, b_spec,
        ],
        out_specs=[o_spec, o_spec, o_spec],
        out_shape=[out_sd, out_sd, out_sd],
    )(x, Wq, Wk, Wv, bq.reshape(2, Dh), bk.reshape(2, Dh), bv.reshape(2, Dh))


def _edge_body(nodes_n, chunk_c, chunks_n, scale,
               k2, q2, v2, src, dst, zout, sout,
               src_i, dst_i, src_o, dst_o, kb, qb, vb, mb, mvb, zb,
               zacc, sacc, sem):
    N = nodes_n
    C = chunk_c
    Dh = kb.shape[1]
    NS = 16
    rows_per = N // NS
    ZB = zb.shape[0]
    Es = chunks_n * C

    c = lax.axis_index("c")
    s = lax.axis_index("s")
    cN = c * N

    def zfill(i, _):
        for j in range(Dh // 16):
            zb[i, pl.ds(j * 16, 16)] = jnp.zeros((16,), jnp.float32)
        return 0
    lax.fori_loop(0, ZB, zfill, 0)
    for r in range(rows_per // ZB):
        base = s * rows_per + r * ZB
        pltpu.sync_copy(zb, zacc.at[pl.ds(base, ZB)])
        pltpu.sync_copy(zb, sacc.at[pl.ds(base, ZB)])
    plsc.subcore_barrier()

    def chunk(ch, _):
        ebase = s * Es + ch * C
        pltpu.sync_copy(src.at[pl.ds(ebase, C)], src_i)
        pltpu.sync_copy(dst.at[pl.ds(ebase, C)], dst_i)
        for t in range(C // 16):
            sl = pl.ds(t * 16, 16)
            src_o[sl] = src_i[sl] + cN
            dst_o[sl] = dst_i[sl] + cN
        cp1 = pltpu.async_copy(k2.at[src_o], kb, sem)
        cp2 = pltpu.async_copy(q2.at[dst_o], qb, sem)
        cp3 = pltpu.async_copy(v2.at[src_o], vb, sem)
        cp1.wait()
        cp2.wait()
        cp3.wait()

        def edge(e, _):
            for j in range(Dh // 16):
                sl = pl.ds(j * 16, 16)
                mm = jnp.exp(kb[e, sl] * qb[e, sl] * scale)
                mb[e, sl] = mm
                mvb[e, sl] = mm * vb[e, sl]
            return 0
        lax.fori_loop(0, C, edge, 0)

        pltpu.sync_copy(mb, zacc.at[dst_i], add=True)
        pltpu.sync_copy(mvb, sacc.at[dst_i], add=True)
        return 0
    lax.fori_loop(0, chunks_n, chunk, 0)
    plsc.subcore_barrier()

    wbase = s * rows_per
    pltpu.sync_copy(zacc.at[pl.ds(wbase, rows_per)],
                    zout.at[pl.ds(cN + wbase, rows_per)])
    pltpu.sync_copy(sacc.at[pl.ds(wbase, rows_per)],
                    sout.at[pl.ds(cN + wbase, rows_per)])


def _edge_pass(k2, q2, v2, src, dst, N, Dh, dk):
    E = src.shape[0]
    NS = 16
    Es = E // NS
    C = 80
    nch = Es // C
    ZB = 125
    scale = 1.0 / math.sqrt(dk)
    mesh = plsc.VectorSubcoreMesh(core_axis_name="c", subcore_axis_name="s")
    out_sd = jax.ShapeDtypeStruct((2 * N, Dh), jnp.float32)
    f = pl.kernel(
        functools.partial(_edge_body, N, C, nch, scale),
        out_type=(out_sd, out_sd),
        mesh=mesh,
        scratch_types=[
            pltpu.VMEM((C,), jnp.int32),
            pltpu.VMEM((C,), jnp.int32),
            pltpu.VMEM((C,), jnp.int32),
            pltpu.VMEM((C,), jnp.int32),
            pltpu.VMEM((C, Dh), jnp.float32),
            pltpu.VMEM((C, Dh), jnp.float32),
            pltpu.VMEM((C, Dh), jnp.float32),
            pltpu.VMEM((C, Dh), jnp.float32),
            pltpu.VMEM((C, Dh), jnp.float32),
            pltpu.VMEM((125, Dh), jnp.float32),
            pltpu.VMEM_SHARED((N, Dh), jnp.float32),
            pltpu.VMEM_SHARED((N, Dh), jnp.float32),
            pltpu.SemaphoreType.DMA,
        ],
    )
    return f(k2, q2, v2, src, dst)


def _out_body(zl, zh, sl_, sh, wo, bo, out):
    za = zl[0]
    zb = zh[0]
    rl = sl_[0] / jnp.where(za == 0.0, 1.0, za)
    rh = sh[0] / jnp.where(zb == 0.0, 1.0, zb)
    r = jnp.concatenate([rl, rh], axis=1)
    dn = (((1,), (1,)), ((), ()))
    out[...] = lax.dot_general(r, wo[...], dn, preferred_element_type=jnp.float32) + bo[...]


def _out_proj(Z, S, Wo, bo, N, D):
    Dh = D // 2
    B = 625
    nb = N // B
    z3 = Z.reshape(2, N, Dh)
    s3 = S.reshape(2, N, Dh)
    lo_spec = pl.BlockSpec((1, B, Dh), lambda i: (0, i, 0))
    hi_spec = pl.BlockSpec((1, B, Dh), lambda i: (1, i, 0))
    return pl.pallas_call(
        _out_body,
        grid=(nb,),
        in_specs=[
            lo_spec, hi_spec, lo_spec, hi_spec,
            pl.BlockSpec((D, D), lambda i: (0, 0)),
            pl.BlockSpec((1, D), lambda i: (0, 0)),
        ],
        out_specs=pl.BlockSpec((B, D), lambda i: (i, 0)),
        out_shape=jax.ShapeDtypeStruct((N, D), jnp.float32),
    )(z3, z3, s3, s3, Wo, bo.reshape(1, D))


def kernel(x, edge_index, Wq, bq, Wk, bk, Wv, bv, Wo, bo):
    N, D = x.shape
    Dh = D // 2
    dk = D // H
    q2, k2, v2 = _qkv_proj(x, Wq, bq, Wk, bk, Wv, bv)
    src = edge_index[0]
    dst = edge_index[1]
    Z, S = _edge_pass(k2, q2, v2, src, dst, N, Dh, dk)
    return _out_proj(Z, S, Wo, bo, N, D)


# speedup vs baseline: 85.7016x; 85.7016x over previous
import functools
import math

import jax
import jax.numpy as jnp
from jax import lax
from jax.experimental import pallas as pl
from jax.experimental.pallas import tpu as pltpu
from jax.experimental.pallas import tpu_sc as plsc

H = 8


def _qkv_body(xb, wq, wk, wv, bq, bk, bv, qo, ko, vo):
    x = xb[...]
    dn = (((1,), (1,)), ((), ()))
    qo[...] = lax.dot_general(x, wq[...], dn, preferred_element_type=jnp.float32) + bq[0]
    ko[...] = lax.dot_general(x, wk[...], dn, preferred_element_type=jnp.float32) + bk[0]
    vo[...] = lax.dot_general(x, wv[...], dn, preferred_element_type=jnp.float32) + bv[0]


def _qkv_proj(x, Wq, bq, Wk, bk, Wv, bv):
    N, D = x.shape
    Dh = D // 2
    B = 1000
    nb = N // B
    out_sd = jax.ShapeDtypeStruct((2 * N, Dh), jnp.float32)
    w_spec = pl.BlockSpec((Dh, D), lambda i, h: (h, 0))
    b_spec = pl.BlockSpec((1, 1, Dh), lambda i, h: (h, 0, 0))
    o_spec = pl.BlockSpec((B, Dh), lambda i, h: (h * nb + i, 0))
    return pl.pallas_call(
        _qkv_body,
        grid=(nb, 2),
        in_specs=[
            pl.BlockSpec((B, D), lambda i, h: (i, 0)),
            w_spec, w_spec, w_spec, b_spec, b_spec, b_spec,
        ],
        out_specs=[o_spec, o_spec, o_spec],
        out_shape=[out_sd, out_sd, out_sd],
    )(x, Wq, Wk, Wv, bq.reshape(2, 1, Dh), bk.reshape(2, 1, Dh), bv.reshape(2, 1, Dh))


def _edge_body(nodes_n, chunk_c, chunks_n, scale,
               k2, q2, v2, src, dst, zout, sout,
               src_i, dst_i, src_o, dst_o, kb, qb, vb, mb, mvb, zb,
               zacc, sacc, sem):
    N = nodes_n
    C = chunk_c
    Dh = kb.shape[1]
    NW = 10
    rows_per = N // NW
    ZB = zb.shape[0]
    Es = chunks_n * C

    c = lax.axis_index("c")
    s = lax.axis_index("s")
    cN = c * N

    def zfill(i, _):
        for j in range(Dh // 16):
            zb[i, pl.ds(j * 16, 16)] = jnp.zeros((16,), jnp.float32)
        return 0
    lax.fori_loop(0, ZB, zfill, 0)

    @pl.when(s < NW)
    def _zero():
        for r in range(rows_per // ZB):
            base = s * rows_per + r * ZB
            pltpu.sync_copy(zb, zacc.at[pl.ds(base, ZB)])
            pltpu.sync_copy(zb, sacc.at[pl.ds(base, ZB)])
    plsc.subcore_barrier()

    def chunk(ch, _):
        ebase = s * Es + ch * C
        pltpu.sync_copy(src.at[pl.ds(ebase, C)], src_i)
        pltpu.sync_copy(dst.at[pl.ds(ebase, C)], dst_i)
        for t in range(C // 16):
            sl = pl.ds(t * 16, 16)
            src_o[sl] = src_i[sl] + cN
            dst_o[sl] = dst_i[sl] + cN
        cp1 = pltpu.async_copy(k2.at[src_o], kb, sem)
        cp2 = pltpu.async_copy(q2.at[dst_o], qb, sem)
        cp3 = pltpu.async_copy(v2.at[src_o], vb, sem)
        cp1.wait()
        cp2.wait()
        cp3.wait()

        def edge(e, _):
            for j in range(Dh // 16):
                sl = pl.ds(j * 16, 16)
                mm = jnp.exp(kb[e, sl] * qb[e, sl] * scale)
                mb[e, sl] = mm
                mvb[e, sl] = mm * vb[e, sl]
            return 0
        lax.fori_loop(0, C, edge, 0)

        pltpu.sync_copy(mb, zacc.at[dst_i], add=True)
        pltpu.sync_copy(mvb, sacc.at[dst_i], add=True)
        return 0
    lax.fori_loop(0, chunks_n, chunk, 0)
    plsc.subcore_barrier()

    @pl.when(s < NW)
    def _writeout():
        wbase = s * rows_per
        pltpu.sync_copy(zacc.at[pl.ds(wbase, rows_per)],
                        zout.at[pl.ds(cN + wbase, rows_per)])
        pltpu.sync_copy(sacc.at[pl.ds(wbase, rows_per)],
                        sout.at[pl.ds(cN + wbase, rows_per)])


def _edge_pass(k2, q2, v2, src, dst, N, Dh, dk):
    E = src.shape[0]
    NS = 16
    Es = E // NS
    C = 80
    nch = Es // C
    ZB = 200
    scale = 1.0 / math.sqrt(dk)
    mesh = plsc.VectorSubcoreMesh(core_axis_name="c", subcore_axis_name="s")
    out_sd = jax.ShapeDtypeStruct((2 * N, Dh), jnp.float32)
    f = pl.kernel(
        functools.partial(_edge_body, N, C, nch, scale),
        out_type=(out_sd, out_sd),
        mesh=mesh,
        scratch_types=[
            pltpu.VMEM((C,), jnp.int32),
            pltpu.VMEM((C,), jnp.int32),
            pltpu.VMEM((C,), jnp.int32),
            pltpu.VMEM((C,), jnp.int32),
            pltpu.VMEM((C, Dh), jnp.float32),
            pltpu.VMEM((C, Dh), jnp.float32),
            pltpu.VMEM((C, Dh), jnp.float32),
            pltpu.VMEM((C, Dh), jnp.float32),
            pltpu.VMEM((C, Dh), jnp.float32),
            pltpu.VMEM((200, Dh), jnp.float32),
            pltpu.VMEM_SHARED((N, Dh), jnp.float32),
            pltpu.VMEM_SHARED((N, Dh), jnp.float32),
            pltpu.SemaphoreType.DMA,
        ],
        compiler_params=pltpu.CompilerParams(use_tc_tiling_on_sc=False),
    )
    return f(k2, q2, v2, src, dst)


def _out_body(zl, zh, sl_, sh, wo, bo, out):
    za = zl[0]
    zb = zh[0]
    rl = sl_[0] / jnp.where(za == 0.0, 1.0, za)
    rh = sh[0] / jnp.where(zb == 0.0, 1.0, zb)
    r = jnp.concatenate([rl, rh], axis=1)
    dn = (((1,), (1,)), ((), ()))
    out[...] = lax.dot_general(r, wo[...], dn, preferred_element_type=jnp.float32) + bo[...]


def _out_proj(Z, S, Wo, bo, N, D):
    Dh = D // 2
    B = 1000
    nb = N // B
    z3 = Z.reshape(2, N, Dh)
    s3 = S.reshape(2, N, Dh)
    lo_spec = pl.BlockSpec((1, B, Dh), lambda i: (0, i, 0))
    hi_spec = pl.BlockSpec((1, B, Dh), lambda i: (1, i, 0))
    return pl.pallas_call(
        _out_body,
        grid=(nb,),
        in_specs=[
            lo_spec, hi_spec, lo_spec, hi_spec,
            pl.BlockSpec((D, D), lambda i: (0, 0)),
            pl.BlockSpec((1, D), lambda i: (0, 0)),
        ],
        out_specs=pl.BlockSpec((B, D), lambda i: (i, 0)),
        out_shape=jax.ShapeDtypeStruct((N, D), jnp.float32),
    )(z3, z3, s3, s3, Wo, bo.reshape(1, D))


def kernel(x, edge_index, Wq, bq, Wk, bk, Wv, bv, Wo, bo):
    N, D = x.shape
    Dh = D // 2
    dk = D // H
    q2, k2, v2 = _qkv_proj(x, Wq, bq, Wk, bk, Wv, bv)
    src = edge_index[0]
    dst = edge_index[1]
    Z, S = _edge_pass(k2, q2, v2, src, dst, N, Dh, dk)
    return _out_proj(Z, S, Wo, bo, N, D)
